# baseline (speedup 1.0000x reference)
import jax
import jax.numpy as jnp
from jax import lax
from jax.experimental import pallas as pl
from jax.experimental.pallas import tpu as pltpu
from jax.experimental.pallas import tpu_sc as plsc

NC = 2
NS = 16
NW = NC * NS


def _make_sc_broadcast(B, S, D, chunk, nbuf):
    rows_per_w = S // NW
    n_chunks = rows_per_w // chunk
    mesh = plsc.VectorSubcoreMesh(core_axis_name="c", subcore_axis_name="s")

    spmem_batch = B - 1
    sch = chunk // 2
    n_srounds = rows_per_w // sch

    def body(table_hbm, out_hbm, *rest):
        bufs, (shared, rsem, wsem, s_rsem, s_wsem) = rest[:nbuf], rest[nbuf:]
        sid = lax.axis_index("s")
        wid = sid * NC + lax.axis_index("c")
        base = wid * rows_per_w

        def sread(r):
            return pltpu.make_async_copy(
                table_hbm.at[pl.ds(base + r * sch, sch)],
                shared.at[sid, r % 2],
                s_rsem,
            )

        def swrite(r):
            return pltpu.make_async_copy(
                shared.at[sid, r % 2],
                out_hbm.at[spmem_batch, pl.ds(base + r * sch, sch)],
                s_wsem,
            )

        def read(c):
            return pltpu.make_async_copy(
                table_hbm.at[pl.ds(base + c * chunk, chunk)], bufs[c % nbuf], rsem
            )

        def writes(c):
            return [
                pltpu.make_async_copy(
                    bufs[c % nbuf], out_hbm.at[b, pl.ds(base + c * chunk, chunk)], wsem
                )
                for b in range(spmem_batch)
            ]

        k = chunk // sch
        sread(0).start()
        for c in range(min(nbuf, n_chunks)):
            read(c).start()
        drained = 0
        for c in range(n_chunks):
            for r in range(c * k, (c + 1) * k):
                if r >= 1:
                    swrite(r - 1).wait()
                sread(r).wait()
                if r + 1 < n_srounds:
                    sread(r + 1).start()
                swrite(r).start()
            read(c).wait()
            nxt = c - 1 + nbuf
            if c >= 1 and nxt < n_chunks:
                for cp in writes(c - 1):
                    cp.wait()
                drained = c
                read(nxt).start()
            for cp in writes(c):
                cp.start()
        for c in range(drained, n_chunks):
            for cp in writes(c):
                cp.wait()
        swrite(n_srounds - 1).wait()

    return pl.kernel(
        body,
        out_type=jax.ShapeDtypeStruct((B, S, D), jnp.float32),
        mesh=mesh,
        scratch_types=[pltpu.VMEM((chunk, D), jnp.float32)] * nbuf
        + [
            pltpu.VMEM_SHARED((NS, 2, sch, D), jnp.float32),
            pltpu.SemaphoreType.DMA,
            pltpu.SemaphoreType.DMA,
            pltpu.SemaphoreType.DMA,
            pltpu.SemaphoreType.DMA,
        ],
    )


def kernel(x, pos_table):
    B, S, D = x.shape
    return _make_sc_broadcast(B, S, D, chunk=32, nbuf=2)(pos_table)
